# baseline (speedup 1.0000x reference)
import functools

import jax
import jax.numpy as jnp
from jax import lax
from jax.experimental import pallas as pl
from jax.experimental.pallas import tpu as pltpu
from jax.experimental.pallas import tpu_sc as plsc

N = 10000
E = 320000
DIN = 128
DH = 256
G = 64

NC = 2
NS = 16
L = 16

EP = 327680
EB = EP // 128
NP2 = 10240
NPAD = NP2
RPT = NP2 // NS
BR = 1000
NB = N // BR

@functools.cache
def _mesh():
    return plsc.VectorSubcoreMesh(core_axis_name="c", subcore_axis_name="s",
                                  num_cores=NC, num_subcores=NS)


def _deg_body(dst2d, out_hbm, dst_v, ones_v, acc_sh):
    c = lax.axis_index("c")
    s = lax.axis_index("s")
    w = c * NS + s
    nblk = EB // (NC * NS)
    pltpu.sync_copy(dst2d.at[pl.ds(w * nblk, nblk)], dst_v)
    init = jnp.where(c == 0, 1.0, 0.0).astype(jnp.float32)
    for r in range(128):
        ones_v[r, :] = jnp.zeros((L,), jnp.float32) + init
    for kk in range(RPT // 128):
        pltpu.sync_copy(ones_v, acc_sh.at[pl.ds(s * RPT + kk * 128, 128)])
    for r in range(128):
        ones_v[r, :] = jnp.ones((L,), jnp.float32)
    plsc.subcore_barrier()

    def body(j, carry):
        pltpu.sync_copy(ones_v, acc_sh.at[dst_v.at[j]], add=True)
        return carry

    lax.fori_loop(0, nblk, body, 0)
    plsc.subcore_barrier()
    pltpu.sync_copy(acc_sh.at[pl.ds(s * RPT, RPT)],
                    out_hbm.at[pl.ds(c * NP2 + s * RPT, RPT)])


@functools.cache
def _deg_kernel():
    return pl.kernel(
        _deg_body,
        out_type=jax.ShapeDtypeStruct((2 * NP2, L), jnp.float32),
        mesh=_mesh(),
        scratch_types=[
            pltpu.VMEM((EB // (NC * NS), 128), jnp.int32),
            pltpu.VMEM((128, L), jnp.float32),
            pltpu.VMEM_SHARED((NP2, L), jnp.float32),
        ],
        compiler_params=pltpu.CompilerParams(needs_layout_passes=False),
    )


BLKE = 64
NRING = 4
EB64 = EP // BLKE
CH64 = 64


def _edge_sweep(t2d, src2d, dst2d, acc_sh, src_v, dst_v, bufs, sem_g,
                sem_s, base, nblk, ch_rows):
    def chunk(ch, carry):
        pltpu.sync_copy(src2d.at[pl.ds(base + ch * ch_rows, ch_rows)], src_v)
        pltpu.sync_copy(dst2d.at[pl.ds(base + ch * ch_rows, ch_rows)], dst_v)
        d_g = [pltpu.async_copy(t2d.at[src_v.at[b]], bufs[b], sem_g[b])
               for b in range(NRING)]
        d_s = [None] * NRING
        for j in range(ch_rows):
            p = j % NRING
            d_g[p].wait()
            d_s[p] = pltpu.async_copy(bufs[p], acc_sh.at[dst_v.at[j]],
                                      sem_s[p], add=True)
            jq = j - 1
            if jq >= 0 and jq + NRING < ch_rows:
                q = jq % NRING
                d_s[q].wait()
                d_g[q] = pltpu.async_copy(t2d.at[src_v.at[jq + NRING]],
                                          bufs[q], sem_g[q])
        for j in range(max(0, ch_rows - NRING), ch_rows):
            d_s[j % NRING].wait()
        return carry

    lax.fori_loop(0, nblk // ch_rows, chunk, 0)


def _agg_scratch(ch):
    return [
        pltpu.VMEM((ch, BLKE), jnp.int32),
        pltpu.VMEM((ch, BLKE), jnp.int32),
    ] + [pltpu.VMEM((BLKE, 128), jnp.float32)] * NRING + [
        pltpu.VMEM_SHARED((NP2, 128), jnp.float32),
    ] + [pltpu.SemaphoreType.DMA] * (2 * NRING)


def _agg_es_body(t2d, src2d, dst2d, out_hbm, *scr):
    src_v, dst_v = scr[0], scr[1]
    bufs = scr[2:2 + NRING]
    acc_sh = scr[2 + NRING]
    sem_g = scr[3 + NRING:3 + 2 * NRING]
    sem_s = scr[3 + 2 * NRING:]
    c = lax.axis_index("c")
    s = lax.axis_index("s")
    w = c * NS + s
    nblk = EB64 // (NC * NS)

    @pl.when(c == 0)
    def _():
        pltpu.sync_copy(t2d.at[pl.ds(s * RPT, RPT)],
                        acc_sh.at[pl.ds(s * RPT, RPT)])

    @pl.when(c == 1)
    def _():
        for kk in range(BLKE * 128 // L):
            bufs[0][kk // (128 // L), pl.ds((kk % (128 // L)) * L, L)] = (
                jnp.zeros((L,), jnp.float32))
        for kk in range(RPT // BLKE):
            pltpu.sync_copy(bufs[0],
                            acc_sh.at[pl.ds(s * RPT + kk * BLKE, BLKE)])

    plsc.subcore_barrier()
    _edge_sweep(t2d, src2d, dst2d, acc_sh, src_v, dst_v, bufs,
                sem_g, sem_s, w * nblk, nblk, 32)
    plsc.subcore_barrier()
    pltpu.sync_copy(acc_sh.at[pl.ds(s * RPT, RPT)],
                    out_hbm.at[pl.ds(c * NP2 + s * RPT, RPT)])


@functools.cache
def _agg_es_kernel():
    return pl.kernel(
        _agg_es_body,
        out_type=jax.ShapeDtypeStruct((2 * NP2, 128), jnp.float32),
        mesh=_mesh(),
        scratch_types=_agg_scratch(32),
        compiler_params=pltpu.CompilerParams(needs_layout_passes=False),
    )


def _agg_body(t_lo, t_hi, src2d, dst2d, out_hbm, *scr):
    src_v, dst_v = scr[0], scr[1]
    bufs = scr[2:2 + NRING]
    acc_sh = scr[2 + NRING]
    sem_g = scr[3 + NRING:3 + 2 * NRING]
    sem_s = scr[3 + 2 * NRING:]
    c = lax.axis_index("c")
    s = lax.axis_index("s")
    nblk = EB64 // NS

    @pl.when(c == 0)
    def _():
        pltpu.sync_copy(t_lo.at[pl.ds(s * RPT, RPT)],
                        acc_sh.at[pl.ds(s * RPT, RPT)])
        plsc.subcore_barrier()
        _edge_sweep(t_lo, src2d, dst2d, acc_sh, src_v, dst_v, bufs,
                    sem_g, sem_s, s * nblk, nblk, CH64)

    @pl.when(c == 1)
    def _():
        pltpu.sync_copy(t_hi.at[pl.ds(s * RPT, RPT)],
                        acc_sh.at[pl.ds(s * RPT, RPT)])
        plsc.subcore_barrier()
        _edge_sweep(t_hi, src2d, dst2d, acc_sh, src_v, dst_v, bufs,
                    sem_g, sem_s, s * nblk, nblk, CH64)

    plsc.subcore_barrier()
    pltpu.sync_copy(acc_sh.at[pl.ds(s * RPT, RPT)],
                    out_hbm.at[pl.ds(c * NP2 + s * RPT, RPT)])


@functools.cache
def _agg_kernel():
    return pl.kernel(
        _agg_body,
        out_type=jax.ShapeDtypeStruct((2 * NP2, 128), jnp.float32),
        mesh=_mesh(),
        scratch_types=_agg_scratch(CH64),
        compiler_params=pltpu.CompilerParams(needs_layout_passes=False),
    )


def _pool_body(t4_hbm, dinv_hbm, batch_hbm, srcp, dstp, out_hbm,
               t4_v, dinv_v, batch_v, src_v, dst_v, acc, red, sh2, sh_red):
    c = lax.axis_index("c")
    s = lax.axis_index("s")
    w = c * NS + s
    ept = EP // (NC * NS)
    pltpu.sync_copy(t4_hbm, t4_v)
    pltpu.sync_copy(dinv_hbm, dinv_v)
    pltpu.sync_copy(batch_hbm, batch_v)
    pltpu.sync_copy(srcp.at[pl.ds(w * ept, ept)], src_v)
    pltpu.sync_copy(dstp.at[pl.ds(w * ept, ept)], dst_v)
    for r in range(L):
        for kk in range(G // L):
            acc[r, pl.ds(kk * L, L)] = jnp.zeros((L,), jnp.float32)
    lanes = lax.iota(jnp.int32, L)

    def body(i, carry):
        s16 = src_v[pl.ds(i * L, L)]
        d16 = dst_v[pl.ds(i * L, L)]
        tv = plsc.load_gather(t4_v, [s16])
        dv = plsc.load_gather(dinv_v, [d16])
        gv = plsc.load_gather(batch_v, [d16])
        plsc.addupdate_scatter(acc, [lanes, gv], tv * dv)
        return carry

    lax.fori_loop(0, ept // L, body, 0)
    for kk in range(G // L):
        tot = jnp.zeros((L,), jnp.float32)
        for r in range(L):
            tot = tot + acc[r, pl.ds(kk * L, L)]
        red[pl.ds(kk * L, L)] = tot
    pltpu.sync_copy(red, sh_red.at[s])
    plsc.subcore_barrier()

    @pl.when(s == 0)
    def _():
        pltpu.sync_copy(sh_red, sh2)
        for kk in range(G // L):
            tot = jnp.zeros((L,), jnp.float32)
            for r in range(NS):
                tot = tot + sh2[r, pl.ds(kk * L, L)]
            red[pl.ds(kk * L, L)] = tot
        pltpu.sync_copy(red, out_hbm.at[c])


@functools.cache
def _pool_kernel():
    return pl.kernel(
        _pool_body,
        out_type=jax.ShapeDtypeStruct((NC, G), jnp.float32),
        mesh=_mesh(),
        scratch_types=[
        pltpu.VMEM((N,), jnp.float32),
        pltpu.VMEM((NPAD,), jnp.float32),
        pltpu.VMEM((NPAD,), jnp.int32),
        pltpu.VMEM((EP // (NC * NS),), jnp.int32),
        pltpu.VMEM((EP // (NC * NS),), jnp.int32),
        pltpu.VMEM((L, G), jnp.float32),
        pltpu.VMEM((G,), jnp.float32),
        pltpu.VMEM((NS, G), jnp.float32),
        pltpu.VMEM_SHARED((NS, G), jnp.float32),
    ],
        compiler_params=pltpu.CompilerParams(needs_layout_passes=False),
    )


def _prep_body(deg_ref, x_ref, dinv_ref, t0_ref):
    p = deg_ref[...]
    deg = p[0, :, 0:1] + p[1, :, 0:1]
    dv = lax.rsqrt(deg)
    dinv_ref[...] = dv
    t0_ref[...] = x_ref[...] * dv


def _prep_call(degparts, x):
    return pl.pallas_call(
        _prep_body,
        grid=(NB,),
        in_specs=[
            pl.BlockSpec((2, BR, L), lambda i: (0, i, 0)),
            pl.BlockSpec((BR, DIN), lambda i: (i, 0)),
        ],
        out_specs=[
            pl.BlockSpec((BR, 1), lambda i: (i, 0)),
            pl.BlockSpec((BR, 128), lambda i: (i, 0)),
        ],
        out_shape=[
            jax.ShapeDtypeStruct((N, 1), jnp.float32),
            jax.ShapeDtypeStruct((NP2, 128), jnp.float32),
        ],
    )(degparts, x)


def _layer1_body(agg_ref, dinv_ref, w1_ref, b1_ref, w2_ref, out_ref):
    a = agg_ref[...]
    af = a[0] + a[1]
    dv = dinv_ref[...]
    u = af * dv
    h = jnp.maximum(
        jnp.dot(u, w1_ref[...], preferred_element_type=jnp.float32)
        + b1_ref[...], 0.0)
    t = jnp.dot(h, w2_ref[...], preferred_element_type=jnp.float32) * dv
    out_ref[0] = t[:, :128]
    out_ref[1] = t[:, 128:]


def _layer1_call(agg0, dinv, W1, b1, W2):
    return pl.pallas_call(
        _layer1_body,
        grid=(NB,),
        in_specs=[
            pl.BlockSpec((2, BR, 128), lambda i: (0, i, 0)),
            pl.BlockSpec((BR, 1), lambda i: (i, 0)),
            pl.BlockSpec((DIN, DH), lambda i: (0, 0)),
            pl.BlockSpec((1, DH), lambda i: (0, 0)),
            pl.BlockSpec((DH, DH), lambda i: (0, 0)),
        ],
        out_specs=pl.BlockSpec((2, BR, 128), lambda i: (0, i, 0)),
        out_shape=jax.ShapeDtypeStruct((2, NP2, 128), jnp.float32),
    )(agg0, dinv, W1, b1, W2)


def _mid_body(agg_ref, dinv_ref, b_ref, w_ref, out_ref):
    a = agg_ref[...]
    af = jnp.concatenate([a[0], a[1]], axis=-1)
    dv = dinv_ref[...]
    h = jnp.maximum(af * dv + b_ref[...], 0.0)
    t = jnp.dot(h, w_ref[...], preferred_element_type=jnp.float32) * dv
    out_ref[0] = t[:, :128]
    out_ref[1] = t[:, 128:]


def _mid_call(agg, dinv, b_prev, W_next):
    return pl.pallas_call(
        _mid_body,
        grid=(NB,),
        in_specs=[
            pl.BlockSpec((2, BR, 128), lambda i: (0, i, 0)),
            pl.BlockSpec((BR, 1), lambda i: (i, 0)),
            pl.BlockSpec((1, DH), lambda i: (0, 0)),
            pl.BlockSpec((DH, DH), lambda i: (0, 0)),
        ],
        out_specs=pl.BlockSpec((2, BR, 128), lambda i: (0, i, 0)),
        out_shape=jax.ShapeDtypeStruct((2, NP2, 128), jnp.float32),
    )(agg, dinv, b_prev, W_next)


def _last_body(agg_ref, dinv_ref, b3_ref, w4_ref, t4_ref, s4_ref):
    a = agg_ref[...]
    af = jnp.concatenate([a[0], a[1]], axis=-1)
    dv = dinv_ref[...]
    h = jnp.maximum(af * dv + b3_ref[...], 0.0)
    y = jnp.dot(h, w4_ref[...], preferred_element_type=jnp.float32)
    t4 = y * dv
    t4_ref[...] = t4
    s4_ref[...] = t4 * dv


def _last_call(agg3, dinv, b3, W4):
    return pl.pallas_call(
        _last_body,
        grid=(NB,),
        in_specs=[
            pl.BlockSpec((2, BR, 128), lambda i: (0, i, 0)),
            pl.BlockSpec((BR, 1), lambda i: (i, 0)),
            pl.BlockSpec((1, DH), lambda i: (0, 0)),
            pl.BlockSpec((DH, 1), lambda i: (0, 0)),
        ],
        out_specs=[
            pl.BlockSpec((BR, 1), lambda i: (i, 0)),
            pl.BlockSpec((BR, 1), lambda i: (i, 0)),
        ],
        out_shape=[
            jax.ShapeDtypeStruct((N, 1), jnp.float32),
            jax.ShapeDtypeStruct((N, 1), jnp.float32),
        ],
    )(agg3, dinv, b3, W4)


def _final_body(batch_ref, s4_ref, ep_ref, b4_ref, out_ref, pool_acc, cnt_acc):
    i = pl.program_id(0)

    @pl.when(i == 0)
    def _():
        pool_acc[...] = jnp.zeros((1, G), jnp.float32)
        cnt_acc[...] = jnp.zeros((1, G), jnp.float32)

    gids = lax.broadcasted_iota(jnp.int32, (BR, G), 1)
    onehot = (batch_ref[...] == gids).astype(jnp.float32)
    pool_acc[...] += jnp.sum(onehot * s4_ref[...], axis=0, keepdims=True)
    cnt_acc[...] += jnp.sum(onehot, axis=0, keepdims=True)

    @pl.when(i == NB - 1)
    def _():
        esum = ep_ref[0:1, :] + ep_ref[1:2, :]
        cnt = cnt_acc[...]
        pooled = (pool_acc[...] + esum + b4_ref[...] * cnt) / jnp.maximum(
            cnt, 1.0)
        out_ref[...] = jax.nn.sigmoid(pooled)


def _final_call(batch2d, s4, edgeparts, b4):
    return pl.pallas_call(
        _final_body,
        grid=(NB,),
        in_specs=[
            pl.BlockSpec((BR, 1), lambda i: (i, 0)),
            pl.BlockSpec((BR, 1), lambda i: (i, 0)),
            pl.BlockSpec((NC, G), lambda i: (0, 0)),
            pl.BlockSpec((1, 1), lambda i: (0, 0)),
        ],
        out_specs=pl.BlockSpec((1, G), lambda i: (0, 0)),
        out_shape=jax.ShapeDtypeStruct((1, G), jnp.float32),
        scratch_shapes=[
            pltpu.VMEM((1, G), jnp.float32),
            pltpu.VMEM((1, G), jnp.float32),
        ],
    )(batch2d, s4, edgeparts, b4)


def kernel(x, edge_index, batch, W1, b1, W2, b2, W3, b3, W4, b4):
    src = edge_index[0]
    dst = edge_index[1]
    npad = EP - E
    pad_src = jnp.arange(npad, dtype=jnp.int32)
    pad_dst = N + pad_src % (NP2 - N)
    src_p = jnp.concatenate([src, pad_src])
    dst_p = jnp.concatenate([dst, pad_dst])
    dst2d = dst_p.reshape(EB, 128)
    src64_0 = src_p.reshape(EB64, BLKE)
    dst64 = dst_p.reshape(EB64, BLKE)

    degparts = _deg_kernel()(dst2d)
    dinv, t0 = _prep_call(degparts.reshape(2, NP2, L), x)

    agg0 = _agg_es_kernel()(t0, src64_0, dst64)
    t2 = _layer1_call(agg0.reshape(2, NP2, 128), dinv, W1,
                      b1.reshape(1, DH), W2)
    agg2 = _agg_kernel()(t2[0], t2[1], src64_0, dst64)
    t3 = _mid_call(agg2.reshape(2, NP2, 128), dinv, b2.reshape(1, DH), W3)
    agg3 = _agg_kernel()(t3[0], t3[1], src64_0, dst64)
    t4, s4 = _last_call(agg3.reshape(2, NP2, 128), dinv, b3.reshape(1, DH),
                        W4)

    dinv_p = jnp.concatenate([dinv.reshape(-1),
                              jnp.zeros((NPAD - N,), jnp.float32)])
    batch_p = jnp.concatenate([batch, jnp.zeros((NPAD - N,), jnp.int32)])
    edgeparts = _pool_kernel()(t4.reshape(-1), dinv_p, batch_p, src_p, dst_p)

    out = _final_call(batch.reshape(N, 1), s4, edgeparts,
                      b4.reshape(1, 1))
    return out.reshape(-1)


# speedup vs baseline: 1.0288x; 1.0288x over previous
import functools

import jax
import jax.numpy as jnp
from jax import lax
from jax.experimental import pallas as pl
from jax.experimental.pallas import tpu as pltpu
from jax.experimental.pallas import tpu_sc as plsc

N = 10000
E = 320000
DIN = 128
DH = 256
G = 64

NC = 2
NS = 16
L = 16

EP = 327680
EB = EP // 128
NP2 = 10240
NPAD = NP2
RPT = NP2 // NS
BR = 1000
NB = N // BR

@functools.cache
def _mesh():
    return plsc.VectorSubcoreMesh(core_axis_name="c", subcore_axis_name="s",
                                  num_cores=NC, num_subcores=NS)


def _deg_body(dst2d, out_hbm, dst_v, ones_v, acc_sh):
    c = lax.axis_index("c")
    s = lax.axis_index("s")
    w = c * NS + s
    nblk = EB // (NC * NS)
    pltpu.sync_copy(dst2d.at[pl.ds(w * nblk, nblk)], dst_v)
    init = jnp.where(c == 0, 1.0, 0.0).astype(jnp.float32)
    for r in range(128):
        ones_v[r, :] = jnp.zeros((L,), jnp.float32) + init
    for kk in range(RPT // 128):
        pltpu.sync_copy(ones_v, acc_sh.at[pl.ds(s * RPT + kk * 128, 128)])
    for r in range(128):
        ones_v[r, :] = jnp.ones((L,), jnp.float32)
    plsc.subcore_barrier()

    def body(j, carry):
        pltpu.sync_copy(ones_v, acc_sh.at[dst_v.at[j]], add=True)
        return carry

    lax.fori_loop(0, nblk, body, 0)
    plsc.subcore_barrier()
    pltpu.sync_copy(acc_sh.at[pl.ds(s * RPT, RPT)],
                    out_hbm.at[pl.ds(c * NP2 + s * RPT, RPT)])


@functools.cache
def _deg_kernel():
    return pl.kernel(
        _deg_body,
        out_type=jax.ShapeDtypeStruct((2 * NP2, L), jnp.float32),
        mesh=_mesh(),
        scratch_types=[
            pltpu.VMEM((EB // (NC * NS), 128), jnp.int32),
            pltpu.VMEM((128, L), jnp.float32),
            pltpu.VMEM_SHARED((NP2, L), jnp.float32),
        ],
        compiler_params=pltpu.CompilerParams(needs_layout_passes=False),
    )


BLKE = 64
NRING = 4
EB64 = EP // BLKE
CH64 = 64
CHES = 32


def _edge_sweep(t2d, src2d, dst2d, acc_sh, src_v, dst_v, bufs, sem_g,
                sem_s, base, nblk, ch_rows):
    def chunk(ch, carry):
        pltpu.sync_copy(src2d.at[pl.ds(base + ch * ch_rows, ch_rows)], src_v)
        pltpu.sync_copy(dst2d.at[pl.ds(base + ch * ch_rows, ch_rows)], dst_v)
        d_g = [pltpu.async_copy(t2d.at[src_v.at[b]], bufs[b], sem_g[b])
               for b in range(NRING)]
        d_s = [None] * NRING
        for j in range(ch_rows):
            p = j % NRING
            d_g[p].wait()
            d_s[p] = pltpu.async_copy(bufs[p], acc_sh.at[dst_v.at[j]],
                                      sem_s[p], add=True)
            jq = j - 1
            if jq >= 0 and jq + NRING < ch_rows:
                q = jq % NRING
                d_s[q].wait()
                d_g[q] = pltpu.async_copy(t2d.at[src_v.at[jq + NRING]],
                                          bufs[q], sem_g[q])
        for j in range(max(0, ch_rows - NRING), ch_rows):
            d_s[j % NRING].wait()
        return carry

    lax.fori_loop(0, nblk // ch_rows, chunk, 0)


def _agg_scratch(ch):
    return [
        pltpu.VMEM((ch, BLKE), jnp.int32),
        pltpu.VMEM((ch, BLKE), jnp.int32),
    ] + [pltpu.VMEM((BLKE, 128), jnp.float32)] * NRING + [
        pltpu.VMEM_SHARED((NP2, 128), jnp.float32),
    ] + [pltpu.SemaphoreType.DMA] * (2 * NRING)


def _agg_es_body(t2d, src2d, dst2d, out_hbm, *scr):
    src_v, dst_v = scr[0], scr[1]
    bufs = scr[2:2 + NRING]
    acc_sh = scr[2 + NRING]
    sem_g = scr[3 + NRING:3 + 2 * NRING]
    sem_s = scr[3 + 2 * NRING:]
    c = lax.axis_index("c")
    s = lax.axis_index("s")
    w = c * NS + s
    nblk = EB64 // (NC * NS)

    @pl.when(c == 0)
    def _():
        pltpu.sync_copy(t2d.at[pl.ds(s * RPT, RPT)],
                        acc_sh.at[pl.ds(s * RPT, RPT)])

    @pl.when(c == 1)
    def _():
        for kk in range(BLKE * 128 // L):
            bufs[0][kk // (128 // L), pl.ds((kk % (128 // L)) * L, L)] = (
                jnp.zeros((L,), jnp.float32))
        for kk in range(RPT // BLKE):
            pltpu.sync_copy(bufs[0],
                            acc_sh.at[pl.ds(s * RPT + kk * BLKE, BLKE)])

    plsc.subcore_barrier()
    _edge_sweep(t2d, src2d, dst2d, acc_sh, src_v, dst_v, bufs,
                sem_g, sem_s, w * nblk, nblk, CHES)
    plsc.subcore_barrier()
    pltpu.sync_copy(acc_sh.at[pl.ds(s * RPT, RPT)],
                    out_hbm.at[pl.ds(c * NP2 + s * RPT, RPT)])


@functools.cache
def _agg_es_kernel():
    return pl.kernel(
        _agg_es_body,
        out_type=jax.ShapeDtypeStruct((2 * NP2, 128), jnp.float32),
        mesh=_mesh(),
        scratch_types=_agg_scratch(CHES),
        compiler_params=pltpu.CompilerParams(needs_layout_passes=False),
    )


def _agg_body(t2d, src0_2d, src1_2d, dst2d, out_hbm, *scr):
    src_v, dst_v = scr[0], scr[1]
    bufs = scr[2:2 + NRING]
    acc_sh = scr[2 + NRING]
    sem_g = scr[3 + NRING:3 + 2 * NRING]
    sem_s = scr[3 + 2 * NRING:]
    c = lax.axis_index("c")
    s = lax.axis_index("s")
    nblk = EB64 // NS
    pltpu.sync_copy(t2d.at[pl.ds(c * NP2 + s * RPT, RPT)],
                    acc_sh.at[pl.ds(s * RPT, RPT)])
    plsc.subcore_barrier()

    @pl.when(c == 0)
    def _():
        _edge_sweep(t2d, src0_2d, dst2d, acc_sh, src_v, dst_v, bufs,
                    sem_g, sem_s, s * nblk, nblk, CH64)

    @pl.when(c == 1)
    def _():
        _edge_sweep(t2d, src1_2d, dst2d, acc_sh, src_v, dst_v, bufs,
                    sem_g, sem_s, s * nblk, nblk, CH64)

    plsc.subcore_barrier()
    pltpu.sync_copy(acc_sh.at[pl.ds(s * RPT, RPT)],
                    out_hbm.at[pl.ds(c * NP2 + s * RPT, RPT)])


@functools.cache
def _agg_kernel():
    return pl.kernel(
        _agg_body,
        out_type=jax.ShapeDtypeStruct((2 * NP2, 128), jnp.float32),
        mesh=_mesh(),
        scratch_types=_agg_scratch(CH64),
        compiler_params=pltpu.CompilerParams(needs_layout_passes=False),
    )


def _pool_body(t4_hbm, dinv_hbm, batch_hbm, srcp, dstp, out_hbm,
               t4_v, dinv_v, batch_v, src_v, dst_v, acc, red, sh2, sh_red):
    c = lax.axis_index("c")
    s = lax.axis_index("s")
    w = c * NS + s
    ept = EP // (NC * NS)
    pltpu.sync_copy(t4_hbm, t4_v)
    pltpu.sync_copy(dinv_hbm, dinv_v)
    pltpu.sync_copy(batch_hbm, batch_v)
    pltpu.sync_copy(srcp.at[pl.ds(w * ept, ept)], src_v)
    pltpu.sync_copy(dstp.at[pl.ds(w * ept, ept)], dst_v)
    for r in range(L):
        for kk in range(G // L):
            acc[r, pl.ds(kk * L, L)] = jnp.zeros((L,), jnp.float32)
    lanes = lax.iota(jnp.int32, L)

    def body(i, carry):
        s16 = src_v[pl.ds(i * L, L)]
        d16 = dst_v[pl.ds(i * L, L)]
        tv = plsc.load_gather(t4_v, [s16])
        dv = plsc.load_gather(dinv_v, [d16])
        gv = plsc.load_gather(batch_v, [d16])
        plsc.addupdate_scatter(acc, [lanes, gv], tv * dv)
        return carry

    lax.fori_loop(0, ept // L, body, 0)
    for kk in range(G // L):
        tot = jnp.zeros((L,), jnp.float32)
        for r in range(L):
            tot = tot + acc[r, pl.ds(kk * L, L)]
        red[pl.ds(kk * L, L)] = tot
    pltpu.sync_copy(red, sh_red.at[s])
    plsc.subcore_barrier()

    @pl.when(s == 0)
    def _():
        pltpu.sync_copy(sh_red, sh2)
        for kk in range(G // L):
            tot = jnp.zeros((L,), jnp.float32)
            for r in range(NS):
                tot = tot + sh2[r, pl.ds(kk * L, L)]
            red[pl.ds(kk * L, L)] = tot
        pltpu.sync_copy(red, out_hbm.at[c])


@functools.cache
def _pool_kernel():
    return pl.kernel(
        _pool_body,
        out_type=jax.ShapeDtypeStruct((NC, G), jnp.float32),
        mesh=_mesh(),
        scratch_types=[
        pltpu.VMEM((N,), jnp.float32),
        pltpu.VMEM((NPAD,), jnp.float32),
        pltpu.VMEM((NPAD,), jnp.int32),
        pltpu.VMEM((EP // (NC * NS),), jnp.int32),
        pltpu.VMEM((EP // (NC * NS),), jnp.int32),
        pltpu.VMEM((L, G), jnp.float32),
        pltpu.VMEM((G,), jnp.float32),
        pltpu.VMEM((NS, G), jnp.float32),
        pltpu.VMEM_SHARED((NS, G), jnp.float32),
    ],
        compiler_params=pltpu.CompilerParams(needs_layout_passes=False),
    )


def _prep_body(deg_ref, x_ref, dinv_ref, t0_ref):
    p = deg_ref[...]
    deg = p[0, :, 0:1] + p[1, :, 0:1]
    dv = lax.rsqrt(deg)
    dinv_ref[...] = dv
    t0_ref[...] = x_ref[...] * dv


def _prep_call(degparts, x):
    return pl.pallas_call(
        _prep_body,
        grid=(NB,),
        in_specs=[
            pl.BlockSpec((2, BR, L), lambda i: (0, i, 0)),
            pl.BlockSpec((BR, DIN), lambda i: (i, 0)),
        ],
        out_specs=[
            pl.BlockSpec((BR, 1), lambda i: (i, 0)),
            pl.BlockSpec((BR, 128), lambda i: (i, 0)),
        ],
        out_shape=[
            jax.ShapeDtypeStruct((N, 1), jnp.float32),
            jax.ShapeDtypeStruct((NP2, 128), jnp.float32),
        ],
    )(degparts, x)


def _layer1_body(agg_ref, dinv_ref, w1_ref, b1_ref, w2_ref, out_ref):
    a = agg_ref[...]
    af = a[0] + a[1]
    dv = dinv_ref[...]
    u = af * dv
    h = jnp.maximum(
        jnp.dot(u, w1_ref[...], preferred_element_type=jnp.float32)
        + b1_ref[...], 0.0)
    t = jnp.dot(h, w2_ref[...], preferred_element_type=jnp.float32) * dv
    out_ref[0] = t[:, :128]
    out_ref[1] = t[:, 128:]


def _layer1_call(agg0, dinv, W1, b1, W2):
    return pl.pallas_call(
        _layer1_body,
        grid=(NB,),
        in_specs=[
            pl.BlockSpec((2, BR, 128), lambda i: (0, i, 0)),
            pl.BlockSpec((BR, 1), lambda i: (i, 0)),
            pl.BlockSpec((DIN, DH), lambda i: (0, 0)),
            pl.BlockSpec((1, DH), lambda i: (0, 0)),
            pl.BlockSpec((DH, DH), lambda i: (0, 0)),
        ],
        out_specs=pl.BlockSpec((2, BR, 128), lambda i: (0, i, 0)),
        out_shape=jax.ShapeDtypeStruct((2, NP2, 128), jnp.float32),
    )(agg0, dinv, W1, b1, W2)


def _mid_body(agg_ref, dinv_ref, b_ref, w_ref, out_ref):
    a = agg_ref[...]
    af = jnp.concatenate([a[0], a[1]], axis=-1)
    dv = dinv_ref[...]
    h = jnp.maximum(af * dv + b_ref[...], 0.0)
    t = jnp.dot(h, w_ref[...], preferred_element_type=jnp.float32) * dv
    out_ref[0] = t[:, :128]
    out_ref[1] = t[:, 128:]


def _mid_call(agg, dinv, b_prev, W_next):
    return pl.pallas_call(
        _mid_body,
        grid=(NB,),
        in_specs=[
            pl.BlockSpec((2, BR, 128), lambda i: (0, i, 0)),
            pl.BlockSpec((BR, 1), lambda i: (i, 0)),
            pl.BlockSpec((1, DH), lambda i: (0, 0)),
            pl.BlockSpec((DH, DH), lambda i: (0, 0)),
        ],
        out_specs=pl.BlockSpec((2, BR, 128), lambda i: (0, i, 0)),
        out_shape=jax.ShapeDtypeStruct((2, NP2, 128), jnp.float32),
    )(agg, dinv, b_prev, W_next)


def _last_body(agg_ref, dinv_ref, b3_ref, w4_ref, t4_ref, s4_ref):
    a = agg_ref[...]
    af = jnp.concatenate([a[0], a[1]], axis=-1)
    dv = dinv_ref[...]
    h = jnp.maximum(af * dv + b3_ref[...], 0.0)
    y = jnp.dot(h, w4_ref[...], preferred_element_type=jnp.float32)
    t4 = y * dv
    t4_ref[...] = t4
    s4_ref[...] = t4 * dv


def _last_call(agg3, dinv, b3, W4):
    return pl.pallas_call(
        _last_body,
        grid=(NB,),
        in_specs=[
            pl.BlockSpec((2, BR, 128), lambda i: (0, i, 0)),
            pl.BlockSpec((BR, 1), lambda i: (i, 0)),
            pl.BlockSpec((1, DH), lambda i: (0, 0)),
            pl.BlockSpec((DH, 1), lambda i: (0, 0)),
        ],
        out_specs=[
            pl.BlockSpec((BR, 1), lambda i: (i, 0)),
            pl.BlockSpec((BR, 1), lambda i: (i, 0)),
        ],
        out_shape=[
            jax.ShapeDtypeStruct((N, 1), jnp.float32),
            jax.ShapeDtypeStruct((N, 1), jnp.float32),
        ],
    )(agg3, dinv, b3, W4)


def _final_body(batch_ref, s4_ref, ep_ref, b4_ref, out_ref, pool_acc, cnt_acc):
    i = pl.program_id(0)

    @pl.when(i == 0)
    def _():
        pool_acc[...] = jnp.zeros((1, G), jnp.float32)
        cnt_acc[...] = jnp.zeros((1, G), jnp.float32)

    gids = lax.broadcasted_iota(jnp.int32, (BR, G), 1)
    onehot = (batch_ref[...] == gids).astype(jnp.float32)
    pool_acc[...] += jnp.sum(onehot * s4_ref[...], axis=0, keepdims=True)
    cnt_acc[...] += jnp.sum(onehot, axis=0, keepdims=True)

    @pl.when(i == NB - 1)
    def _():
        esum = ep_ref[0:1, :] + ep_ref[1:2, :]
        cnt = cnt_acc[...]
        pooled = (pool_acc[...] + esum + b4_ref[...] * cnt) / jnp.maximum(
            cnt, 1.0)
        out_ref[...] = jax.nn.sigmoid(pooled)


def _final_call(batch2d, s4, edgeparts, b4):
    return pl.pallas_call(
        _final_body,
        grid=(NB,),
        in_specs=[
            pl.BlockSpec((BR, 1), lambda i: (i, 0)),
            pl.BlockSpec((BR, 1), lambda i: (i, 0)),
            pl.BlockSpec((NC, G), lambda i: (0, 0)),
            pl.BlockSpec((1, 1), lambda i: (0, 0)),
        ],
        out_specs=pl.BlockSpec((1, G), lambda i: (0, 0)),
        out_shape=jax.ShapeDtypeStruct((1, G), jnp.float32),
        scratch_shapes=[
            pltpu.VMEM((1, G), jnp.float32),
            pltpu.VMEM((1, G), jnp.float32),
        ],
    )(batch2d, s4, edgeparts, b4)


def kernel(x, edge_index, batch, W1, b1, W2, b2, W3, b3, W4, b4):
    src = edge_index[0]
    dst = edge_index[1]
    npad = EP - E
    pad_src = jnp.arange(npad, dtype=jnp.int32)
    pad_dst = N + pad_src % (NP2 - N)
    src_p = jnp.concatenate([src, pad_src])
    dst_p = jnp.concatenate([dst, pad_dst])
    dst2d = dst_p.reshape(EB, 128)
    src64_0 = src_p.reshape(EB64, BLKE)
    src64_1 = src64_0 + NP2
    dst64 = dst_p.reshape(EB64, BLKE)

    degparts = _deg_kernel()(dst2d)
    dinv, t0 = _prep_call(degparts.reshape(2, NP2, L), x)

    agg0 = _agg_es_kernel()(t0, src64_0, dst64)
    t2 = _layer1_call(agg0.reshape(2, NP2, 128), dinv, W1,
                      b1.reshape(1, DH), W2)
    agg2 = _agg_kernel()(t2.reshape(2 * NP2, 128), src64_0, src64_1, dst64)
    t3 = _mid_call(agg2.reshape(2, NP2, 128), dinv, b2.reshape(1, DH), W3)
    agg3 = _agg_kernel()(t3.reshape(2 * NP2, 128), src64_0, src64_1, dst64)
    t4, s4 = _last_call(agg3.reshape(2, NP2, 128), dinv, b3.reshape(1, DH),
                        W4)

    dinv_p = jnp.concatenate([dinv.reshape(-1),
                              jnp.zeros((NPAD - N,), jnp.float32)])
    batch_p = jnp.concatenate([batch, jnp.zeros((NPAD - N,), jnp.int32)])
    edgeparts = _pool_kernel()(t4.reshape(-1), dinv_p, batch_p, src_p, dst_p)

    out = _final_call(batch.reshape(N, 1), s4, edgeparts,
                      b4.reshape(1, 1))
    return out.reshape(-1)


# speedup vs baseline: 1.0395x; 1.0104x over previous
import functools

import jax
import jax.numpy as jnp
from jax import lax
from jax.experimental import pallas as pl
from jax.experimental.pallas import tpu as pltpu
from jax.experimental.pallas import tpu_sc as plsc

N = 10000
E = 320000
DIN = 128
DH = 256
G = 64

NC = 2
NS = 16
L = 16

EP = 327680
EB = EP // 128
NP2 = 10240
NPAD = NP2
RPT = NP2 // NS
BR = 1000
NB = N // BR

@functools.cache
def _mesh():
    return plsc.VectorSubcoreMesh(core_axis_name="c", subcore_axis_name="s",
                                  num_cores=NC, num_subcores=NS)


def _deg_body(dst2d, out_hbm, dst_v, ones_v, acc_sh):
    c = lax.axis_index("c")
    s = lax.axis_index("s")
    w = c * NS + s
    nblk = EB // (NC * NS)
    pltpu.sync_copy(dst2d.at[pl.ds(w * nblk, nblk)], dst_v)
    init = jnp.where(c == 0, 1.0, 0.0).astype(jnp.float32)
    for r in range(128):
        ones_v[r, :] = jnp.zeros((L,), jnp.float32) + init
    for kk in range(RPT // 128):
        pltpu.sync_copy(ones_v, acc_sh.at[pl.ds(s * RPT + kk * 128, 128)])
    for r in range(128):
        ones_v[r, :] = jnp.ones((L,), jnp.float32)
    plsc.subcore_barrier()

    def body(j, carry):
        pltpu.sync_copy(ones_v, acc_sh.at[dst_v.at[j]], add=True)
        return carry

    lax.fori_loop(0, nblk, body, 0)
    plsc.subcore_barrier()
    pltpu.sync_copy(acc_sh.at[pl.ds(s * RPT, RPT)],
                    out_hbm.at[pl.ds(c * NP2 + s * RPT, RPT)])


@functools.cache
def _deg_kernel():
    return pl.kernel(
        _deg_body,
        out_type=jax.ShapeDtypeStruct((2 * NP2, L), jnp.float32),
        mesh=_mesh(),
        scratch_types=[
            pltpu.VMEM((EB // (NC * NS), 128), jnp.int32),
            pltpu.VMEM((128, L), jnp.float32),
            pltpu.VMEM_SHARED((NP2, L), jnp.float32),
        ],
        compiler_params=pltpu.CompilerParams(needs_layout_passes=False),
    )


BLKE = 64
NRING = 5
EB64 = EP // BLKE
CH64 = 32
CHES = 32


def _edge_sweep(t2d, src2d, dst2d, acc_sh, src_v, dst_v, bufs, sem_g,
                sem_s, base, nblk, ch_rows):
    def chunk(ch, carry):
        pltpu.sync_copy(src2d.at[pl.ds(base + ch * ch_rows, ch_rows)], src_v)
        pltpu.sync_copy(dst2d.at[pl.ds(base + ch * ch_rows, ch_rows)], dst_v)
        d_g = [pltpu.async_copy(t2d.at[src_v.at[b]], bufs[b], sem_g[b])
               for b in range(NRING)]
        d_s = [None] * NRING
        for j in range(ch_rows):
            p = j % NRING
            d_g[p].wait()
            d_s[p] = pltpu.async_copy(bufs[p], acc_sh.at[dst_v.at[j]],
                                      sem_s[p], add=True)
            jq = j - 1
            if jq >= 0 and jq + NRING < ch_rows:
                q = jq % NRING
                d_s[q].wait()
                d_g[q] = pltpu.async_copy(t2d.at[src_v.at[jq + NRING]],
                                          bufs[q], sem_g[q])
        for j in range(max(0, ch_rows - NRING), ch_rows):
            d_s[j % NRING].wait()
        return carry

    lax.fori_loop(0, nblk // ch_rows, chunk, 0)


def _agg_scratch(ch):
    return [
        pltpu.VMEM((ch, BLKE), jnp.int32),
        pltpu.VMEM((ch, BLKE), jnp.int32),
    ] + [pltpu.VMEM((BLKE, 128), jnp.float32)] * NRING + [
        pltpu.VMEM_SHARED((NP2, 128), jnp.float32),
    ] + [pltpu.SemaphoreType.DMA] * (2 * NRING)


def _agg_es_body(t2d, src2d, dst2d, out_hbm, *scr):
    src_v, dst_v = scr[0], scr[1]
    bufs = scr[2:2 + NRING]
    acc_sh = scr[2 + NRING]
    sem_g = scr[3 + NRING:3 + 2 * NRING]
    sem_s = scr[3 + 2 * NRING:]
    c = lax.axis_index("c")
    s = lax.axis_index("s")
    w = c * NS + s
    nblk = EB64 // (NC * NS)

    @pl.when(c == 0)
    def _():
        pltpu.sync_copy(t2d.at[pl.ds(s * RPT, RPT)],
                        acc_sh.at[pl.ds(s * RPT, RPT)])

    @pl.when(c == 1)
    def _():
        for kk in range(BLKE * 128 // L):
            bufs[0][kk // (128 // L), pl.ds((kk % (128 // L)) * L, L)] = (
                jnp.zeros((L,), jnp.float32))
        for kk in range(RPT // BLKE):
            pltpu.sync_copy(bufs[0],
                            acc_sh.at[pl.ds(s * RPT + kk * BLKE, BLKE)])

    plsc.subcore_barrier()
    _edge_sweep(t2d, src2d, dst2d, acc_sh, src_v, dst_v, bufs,
                sem_g, sem_s, w * nblk, nblk, CHES)
    plsc.subcore_barrier()
    pltpu.sync_copy(acc_sh.at[pl.ds(s * RPT, RPT)],
                    out_hbm.at[pl.ds(c * NP2 + s * RPT, RPT)])


@functools.cache
def _agg_es_kernel():
    return pl.kernel(
        _agg_es_body,
        out_type=jax.ShapeDtypeStruct((2 * NP2, 128), jnp.float32),
        mesh=_mesh(),
        scratch_types=_agg_scratch(CHES),
        compiler_params=pltpu.CompilerParams(needs_layout_passes=False),
    )


def _agg_body(t2d, src0_2d, src1_2d, dst2d, out_hbm, *scr):
    src_v, dst_v = scr[0], scr[1]
    bufs = scr[2:2 + NRING]
    acc_sh = scr[2 + NRING]
    sem_g = scr[3 + NRING:3 + 2 * NRING]
    sem_s = scr[3 + 2 * NRING:]
    c = lax.axis_index("c")
    s = lax.axis_index("s")
    nblk = EB64 // NS
    pltpu.sync_copy(t2d.at[pl.ds(c * NP2 + s * RPT, RPT)],
                    acc_sh.at[pl.ds(s * RPT, RPT)])
    plsc.subcore_barrier()

    @pl.when(c == 0)
    def _():
        _edge_sweep(t2d, src0_2d, dst2d, acc_sh, src_v, dst_v, bufs,
                    sem_g, sem_s, s * nblk, nblk, CH64)

    @pl.when(c == 1)
    def _():
        _edge_sweep(t2d, src1_2d, dst2d, acc_sh, src_v, dst_v, bufs,
                    sem_g, sem_s, s * nblk, nblk, CH64)

    plsc.subcore_barrier()
    pltpu.sync_copy(acc_sh.at[pl.ds(s * RPT, RPT)],
                    out_hbm.at[pl.ds(c * NP2 + s * RPT, RPT)])


@functools.cache
def _agg_kernel():
    return pl.kernel(
        _agg_body,
        out_type=jax.ShapeDtypeStruct((2 * NP2, 128), jnp.float32),
        mesh=_mesh(),
        scratch_types=_agg_scratch(CH64),
        compiler_params=pltpu.CompilerParams(needs_layout_passes=False),
    )


def _pool_body(t4_hbm, dinv_hbm, batch_hbm, srcp, dstp, out_hbm,
               t4_v, dinv_v, batch_v, src_v, dst_v, acc, red, sh2, sh_red):
    c = lax.axis_index("c")
    s = lax.axis_index("s")
    w = c * NS + s
    ept = EP // (NC * NS)
    pltpu.sync_copy(t4_hbm, t4_v)
    pltpu.sync_copy(dinv_hbm, dinv_v)
    pltpu.sync_copy(batch_hbm, batch_v)
    pltpu.sync_copy(srcp.at[pl.ds(w * ept, ept)], src_v)
    pltpu.sync_copy(dstp.at[pl.ds(w * ept, ept)], dst_v)
    for r in range(L):
        for kk in range(G // L):
            acc[r, pl.ds(kk * L, L)] = jnp.zeros((L,), jnp.float32)
    lanes = lax.iota(jnp.int32, L)

    def body(i, carry):
        s16 = src_v[pl.ds(i * L, L)]
        d16 = dst_v[pl.ds(i * L, L)]
        tv = plsc.load_gather(t4_v, [s16])
        dv = plsc.load_gather(dinv_v, [d16])
        gv = plsc.load_gather(batch_v, [d16])
        plsc.addupdate_scatter(acc, [lanes, gv], tv * dv)
        return carry

    lax.fori_loop(0, ept // L, body, 0)
    for kk in range(G // L):
        tot = jnp.zeros((L,), jnp.float32)
        for r in range(L):
            tot = tot + acc[r, pl.ds(kk * L, L)]
        red[pl.ds(kk * L, L)] = tot
    pltpu.sync_copy(red, sh_red.at[s])
    plsc.subcore_barrier()

    @pl.when(s == 0)
    def _():
        pltpu.sync_copy(sh_red, sh2)
        for kk in range(G // L):
            tot = jnp.zeros((L,), jnp.float32)
            for r in range(NS):
                tot = tot + sh2[r, pl.ds(kk * L, L)]
            red[pl.ds(kk * L, L)] = tot
        pltpu.sync_copy(red, out_hbm.at[c])


@functools.cache
def _pool_kernel():
    return pl.kernel(
        _pool_body,
        out_type=jax.ShapeDtypeStruct((NC, G), jnp.float32),
        mesh=_mesh(),
        scratch_types=[
        pltpu.VMEM((N,), jnp.float32),
        pltpu.VMEM((NPAD,), jnp.float32),
        pltpu.VMEM((NPAD,), jnp.int32),
        pltpu.VMEM((EP // (NC * NS),), jnp.int32),
        pltpu.VMEM((EP // (NC * NS),), jnp.int32),
        pltpu.VMEM((L, G), jnp.float32),
        pltpu.VMEM((G,), jnp.float32),
        pltpu.VMEM((NS, G), jnp.float32),
        pltpu.VMEM_SHARED((NS, G), jnp.float32),
    ],
        compiler_params=pltpu.CompilerParams(needs_layout_passes=False),
    )


def _prep_body(deg_ref, x_ref, dinv_ref, t0_ref):
    p = deg_ref[...]
    deg = p[0, :, 0:1] + p[1, :, 0:1]
    dv = lax.rsqrt(deg)
    dinv_ref[...] = dv
    t0_ref[...] = x_ref[...] * dv


def _prep_call(degparts, x):
    return pl.pallas_call(
        _prep_body,
        grid=(NB,),
        in_specs=[
            pl.BlockSpec((2, BR, L), lambda i: (0, i, 0)),
            pl.BlockSpec((BR, DIN), lambda i: (i, 0)),
        ],
        out_specs=[
            pl.BlockSpec((BR, 1), lambda i: (i, 0)),
            pl.BlockSpec((BR, 128), lambda i: (i, 0)),
        ],
        out_shape=[
            jax.ShapeDtypeStruct((N, 1), jnp.float32),
            jax.ShapeDtypeStruct((NP2, 128), jnp.float32),
        ],
    )(degparts, x)


def _layer1_body(agg_ref, dinv_ref, w1_ref, b1_ref, w2_ref, out_ref):
    a = agg_ref[...]
    af = a[0] + a[1]
    dv = dinv_ref[...]
    u = af * dv
    h = jnp.maximum(
        jnp.dot(u, w1_ref[...], preferred_element_type=jnp.float32)
        + b1_ref[...], 0.0)
    t = jnp.dot(h, w2_ref[...], preferred_element_type=jnp.float32) * dv
    out_ref[0] = t[:, :128]
    out_ref[1] = t[:, 128:]


def _layer1_call(agg0, dinv, W1, b1, W2):
    return pl.pallas_call(
        _layer1_body,
        grid=(NB,),
        in_specs=[
            pl.BlockSpec((2, BR, 128), lambda i: (0, i, 0)),
            pl.BlockSpec((BR, 1), lambda i: (i, 0)),
            pl.BlockSpec((DIN, DH), lambda i: (0, 0)),
            pl.BlockSpec((1, DH), lambda i: (0, 0)),
            pl.BlockSpec((DH, DH), lambda i: (0, 0)),
        ],
        out_specs=pl.BlockSpec((2, BR, 128), lambda i: (0, i, 0)),
        out_shape=jax.ShapeDtypeStruct((2, NP2, 128), jnp.float32),
    )(agg0, dinv, W1, b1, W2)


def _mid_body(agg_ref, dinv_ref, b_ref, w_ref, out_ref):
    a = agg_ref[...]
    af = jnp.concatenate([a[0], a[1]], axis=-1)
    dv = dinv_ref[...]
    h = jnp.maximum(af * dv + b_ref[...], 0.0)
    t = jnp.dot(h, w_ref[...], preferred_element_type=jnp.float32) * dv
    out_ref[0] = t[:, :128]
    out_ref[1] = t[:, 128:]


def _mid_call(agg, dinv, b_prev, W_next):
    return pl.pallas_call(
        _mid_body,
        grid=(NB,),
        in_specs=[
            pl.BlockSpec((2, BR, 128), lambda i: (0, i, 0)),
            pl.BlockSpec((BR, 1), lambda i: (i, 0)),
            pl.BlockSpec((1, DH), lambda i: (0, 0)),
            pl.BlockSpec((DH, DH), lambda i: (0, 0)),
        ],
        out_specs=pl.BlockSpec((2, BR, 128), lambda i: (0, i, 0)),
        out_shape=jax.ShapeDtypeStruct((2, NP2, 128), jnp.float32),
    )(agg, dinv, b_prev, W_next)


def _last_body(agg_ref, dinv_ref, b3_ref, w4_ref, t4_ref, s4_ref):
    a = agg_ref[...]
    af = jnp.concatenate([a[0], a[1]], axis=-1)
    dv = dinv_ref[...]
    h = jnp.maximum(af * dv + b3_ref[...], 0.0)
    y = jnp.dot(h, w4_ref[...], preferred_element_type=jnp.float32)
    t4 = y * dv
    t4_ref[...] = t4
    s4_ref[...] = t4 * dv


def _last_call(agg3, dinv, b3, W4):
    return pl.pallas_call(
        _last_body,
        grid=(NB,),
        in_specs=[
            pl.BlockSpec((2, BR, 128), lambda i: (0, i, 0)),
            pl.BlockSpec((BR, 1), lambda i: (i, 0)),
            pl.BlockSpec((1, DH), lambda i: (0, 0)),
            pl.BlockSpec((DH, 1), lambda i: (0, 0)),
        ],
        out_specs=[
            pl.BlockSpec((BR, 1), lambda i: (i, 0)),
            pl.BlockSpec((BR, 1), lambda i: (i, 0)),
        ],
        out_shape=[
            jax.ShapeDtypeStruct((N, 1), jnp.float32),
            jax.ShapeDtypeStruct((N, 1), jnp.float32),
        ],
    )(agg3, dinv, b3, W4)


def _final_body(batch_ref, s4_ref, ep_ref, b4_ref, out_ref, pool_acc, cnt_acc):
    i = pl.program_id(0)

    @pl.when(i == 0)
    def _():
        pool_acc[...] = jnp.zeros((1, G), jnp.float32)
        cnt_acc[...] = jnp.zeros((1, G), jnp.float32)

    gids = lax.broadcasted_iota(jnp.int32, (BR, G), 1)
    onehot = (batch_ref[...] == gids).astype(jnp.float32)
    pool_acc[...] += jnp.sum(onehot * s4_ref[...], axis=0, keepdims=True)
    cnt_acc[...] += jnp.sum(onehot, axis=0, keepdims=True)

    @pl.when(i == NB - 1)
    def _():
        esum = ep_ref[0:1, :] + ep_ref[1:2, :]
        cnt = cnt_acc[...]
        pooled = (pool_acc[...] + esum + b4_ref[...] * cnt) / jnp.maximum(
            cnt, 1.0)
        out_ref[...] = jax.nn.sigmoid(pooled)


def _final_call(batch2d, s4, edgeparts, b4):
    return pl.pallas_call(
        _final_body,
        grid=(NB,),
        in_specs=[
            pl.BlockSpec((BR, 1), lambda i: (i, 0)),
            pl.BlockSpec((BR, 1), lambda i: (i, 0)),
            pl.BlockSpec((NC, G), lambda i: (0, 0)),
            pl.BlockSpec((1, 1), lambda i: (0, 0)),
        ],
        out_specs=pl.BlockSpec((1, G), lambda i: (0, 0)),
        out_shape=jax.ShapeDtypeStruct((1, G), jnp.float32),
        scratch_shapes=[
            pltpu.VMEM((1, G), jnp.float32),
            pltpu.VMEM((1, G), jnp.float32),
        ],
    )(batch2d, s4, edgeparts, b4)


def kernel(x, edge_index, batch, W1, b1, W2, b2, W3, b3, W4, b4):
    src = edge_index[0]
    dst = edge_index[1]
    npad = EP - E
    pad_src = jnp.arange(npad, dtype=jnp.int32)
    pad_dst = N + pad_src % (NP2 - N)
    src_p = jnp.concatenate([src, pad_src])
    dst_p = jnp.concatenate([dst, pad_dst])
    dst2d = dst_p.reshape(EB, 128)
    src64_0 = src_p.reshape(EB64, BLKE)
    src64_1 = src64_0 + NP2
    dst64 = dst_p.reshape(EB64, BLKE)

    degparts = _deg_kernel()(dst2d)
    dinv, t0 = _prep_call(degparts.reshape(2, NP2, L), x)

    agg0 = _agg_es_kernel()(t0, src64_0, dst64)
    t2 = _layer1_call(agg0.reshape(2, NP2, 128), dinv, W1,
                      b1.reshape(1, DH), W2)
    agg2 = _agg_kernel()(t2.reshape(2 * NP2, 128), src64_0, src64_1, dst64)
    t3 = _mid_call(agg2.reshape(2, NP2, 128), dinv, b2.reshape(1, DH), W3)
    agg3 = _agg_kernel()(t3.reshape(2 * NP2, 128), src64_0, src64_1, dst64)
    t4, s4 = _last_call(agg3.reshape(2, NP2, 128), dinv, b3.reshape(1, DH),
                        W4)

    dinv_p = jnp.concatenate([dinv.reshape(-1),
                              jnp.zeros((NPAD - N,), jnp.float32)])
    batch_p = jnp.concatenate([batch, jnp.zeros((NPAD - N,), jnp.int32)])
    edgeparts = _pool_kernel()(t4.reshape(-1), dinv_p, batch_p, src_p, dst_p)

    out = _final_call(batch.reshape(N, 1), s4, edgeparts,
                      b4.reshape(1, 1))
    return out.reshape(-1)
